# baseline (speedup 1.0000x reference)
import jax
import jax.numpy as jnp
import numpy as np
from jax.experimental import pallas as pl
from jax.experimental.pallas import tpu as pltpu
from jax.experimental.shard_map import shard_map
from jax.sharding import Mesh, PartitionSpec as P


def _gcn_body(x_ref, w_ref, b_ref, adj_ref, out_ref, support_ref):
    @pl.when(pl.program_id(0) == 0)
    def _():
        support_ref[...] = jnp.dot(
            x_ref[...], w_ref[...], preferred_element_type=jnp.float32
        )

    out_ref[...] = (
        jnp.dot(adj_ref[...], support_ref[...], preferred_element_type=jnp.float32)
        + b_ref[...]
    )


def _pallas_gcn(x, adj_rows, w, b2):
    n_rows = adj_rows.shape[0]
    n, d_in = x.shape
    d_out = w.shape[1]
    bm = next(c for c in (400, 200, 100, 40, 8) if n_rows % c == 0)
    return pl.pallas_call(
        _gcn_body,
        grid=(n_rows // bm,),
        in_specs=[
            pl.BlockSpec((n, d_in), lambda i: (0, 0)),
            pl.BlockSpec((d_in, d_out), lambda i: (0, 0)),
            pl.BlockSpec((1, d_out), lambda i: (0, 0)),
            pl.BlockSpec((bm, n), lambda i: (i, 0)),
        ],
        out_specs=pl.BlockSpec((bm, d_out), lambda i: (i, 0)),
        out_shape=jax.ShapeDtypeStruct((n_rows, d_out), jnp.float32),
        scratch_shapes=[pltpu.VMEM((n, d_out), jnp.float32)],
    )(x, w, b2, adj_rows)


def kernel(input, adj, W, b):
    n = adj.shape[0]
    b2 = b.reshape(1, -1)
    devs = jax.devices()
    nd = len(devs)
    while nd > 1 and (n % nd != 0 or (n // nd) % 8 != 0):
        nd -= 1
    if nd == 1:
        return _pallas_gcn(input, adj, W, b2)
    mesh = Mesh(np.array(devs[:nd]), ("d",))
    sharded = shard_map(
        _pallas_gcn,
        mesh=mesh,
        in_specs=(P(), P("d", None), P(), P()),
        out_specs=P("d", None),
        check_rep=False,
    )
    return sharded(input, adj, W, b2)
